# baseline (speedup 1.0000x reference)
import functools

import jax
import jax.numpy as jnp
from jax import lax
from jax.experimental import pallas as pl
from jax.experimental.pallas import tpu as pltpu
from jax.experimental.pallas import tpu_sc as plsc

N = 10000
H = 128
E = 320000
EPS = 1e-5
NC = 2
NS = 16
NW = NC * NS
CH = 128
NSTEP = 80
EPAD = NW * CH * NSTEP
NP = 10112
SLC = NP // NS
NPY = N + 16

_MESH = plsc.VectorSubcoreMesh(
    core_axis_name="c", subcore_axis_name="s", num_cores=NC, num_subcores=NS
)


def _sc_gather_scatter(y_pad, sd_idx, zeros_blk):
    @functools.partial(
        pl.kernel,
        out_type=jax.ShapeDtypeStruct((NC, NP, H), jnp.float32),
        mesh=_MESH,
        scratch_types=[
            pltpu.VMEM((2, CH), jnp.int32),
            pltpu.VMEM((2, CH), jnp.int32),
            pltpu.VMEM((CH, H), jnp.float32),
            pltpu.VMEM((CH, H), jnp.float32),
            pltpu.VMEM_SHARED((NP, H), jnp.float32),
            pltpu.SemaphoreType.DMA,
            pltpu.SemaphoreType.DMA,
            pltpu.SemaphoreType.DMA,
            pltpu.SemaphoreType.DMA,
            pltpu.SemaphoreType.DMA,
            pltpu.SemaphoreType.DMA,
        ],
    )
    def k(y_hbm, sd_hbm, z_hbm, acc_hbm, iA, iB, b0, b1, acc_sh,
          ia, ib, g0, g1, s0, s1):
        c = lax.axis_index("c")
        s = lax.axis_index("s")
        w = c * NS + s
        pltpu.sync_copy(z_hbm, acc_sh.at[pl.ds(s * SLC, SLC)])

        def idx_load(i, buf, sem):
            pltpu.async_copy(sd_hbm.at[w, i], buf, sem)

        def wait_idx(buf, sem):
            pltpu.make_async_copy(sd_hbm.at[0, 0], buf, sem).wait()

        def gather(buf_i, buf, sem):
            pltpu.async_copy(y_hbm.at[buf_i.at[0]], buf, sem)

        def wait_gather(buf, sem):
            pltpu.make_async_copy(y_hbm.at[pl.ds(0, CH)], buf, sem).wait()

        def scatter(buf_i, buf, sem):
            pltpu.async_copy(buf, acc_sh.at[buf_i.at[1]], sem, add=True)

        def wait_scatter(buf, sem):
            pltpu.make_async_copy(buf, acc_sh.at[pl.ds(0, CH)], sem).wait()

        idx_load(0, iA, ia)
        idx_load(1, iB, ib)
        plsc.subcore_barrier()
        wait_idx(iA, ia)
        gather(iA, b0, g0)
        wait_idx(iB, ib)
        gather(iB, b1, g1)

        @pl.loop(0, NSTEP, step=2)
        def _(i):
            wait_gather(b0, g0)
            scatter(iA, b0, s0)
            wait_gather(b1, g1)
            scatter(iB, b1, s1)

            @pl.when(i + 2 < NSTEP)
            def _():
                wait_scatter(b0, s0)
                idx_load(i + 2, iA, ia)
                wait_scatter(b1, s1)
                idx_load(i + 3, iB, ib)
                wait_idx(iA, ia)
                gather(iA, b0, g0)
                wait_idx(iB, ib)
                gather(iB, b1, g1)

        wait_scatter(b0, s0)
        wait_scatter(b1, s1)
        plsc.subcore_barrier()
        pltpu.sync_copy(
            acc_sh.at[pl.ds(s * SLC, SLC)], acc_hbm.at[c, pl.ds(s * SLC, SLC)]
        )

    return k(y_pad, sd_idx, zeros_blk)


def _sc_degree(sd_idx, ones_blk, zeros_blk):
    @functools.partial(
        pl.kernel,
        out_type=jax.ShapeDtypeStruct((NC, NP, H), jnp.float32),
        mesh=_MESH,
        scratch_types=[
            pltpu.VMEM((2, CH), jnp.int32),
            pltpu.VMEM((2, CH), jnp.int32),
            pltpu.VMEM((CH, H), jnp.float32),
            pltpu.VMEM_SHARED((NP, H), jnp.float32),
            pltpu.SemaphoreType.DMA,
            pltpu.SemaphoreType.DMA,
            pltpu.SemaphoreType.DMA,
            pltpu.SemaphoreType.DMA,
        ],
    )
    def k(sd_hbm, o_hbm, z_hbm, acc_hbm, iA, iB, rows, acc_sh, ia, ib, s0, s1):
        c = lax.axis_index("c")
        s = lax.axis_index("s")
        w = c * NS + s
        pltpu.sync_copy(o_hbm, rows)
        pltpu.sync_copy(z_hbm, acc_sh.at[pl.ds(s * SLC, SLC)])

        def idx_load(i, buf, sem):
            pltpu.async_copy(sd_hbm.at[w, i], buf, sem)

        def wait_idx(buf, sem):
            pltpu.make_async_copy(sd_hbm.at[0, 0], buf, sem).wait()

        def scatter(buf_i, sem):
            pltpu.async_copy(rows, acc_sh.at[buf_i.at[1]], sem, add=True)

        def wait_scatter(sem):
            pltpu.make_async_copy(rows, acc_sh.at[pl.ds(0, CH)], sem).wait()

        idx_load(0, iA, ia)
        idx_load(1, iB, ib)
        plsc.subcore_barrier()
        wait_idx(iA, ia)
        scatter(iA, s0)
        wait_idx(iB, ib)
        scatter(iB, s1)

        @pl.loop(0, NSTEP, step=2)
        def _(i):
            @pl.when(i + 2 < NSTEP)
            def _():
                wait_scatter(s0)
                idx_load(i + 2, iA, ia)
                wait_scatter(s1)
                idx_load(i + 3, iB, ib)
                wait_idx(iA, ia)
                scatter(iA, s0)
                wait_idx(iB, ib)
                scatter(iB, s1)

        wait_scatter(s0)
        wait_scatter(s1)
        plsc.subcore_barrier()
        pltpu.sync_copy(
            acc_sh.at[pl.ds(s * SLC, SLC)], acc_hbm.at[c, pl.ds(s * SLC, SLC)]
        )

    return k(sd_idx, ones_blk, zeros_blk)


def _tc_first(x, degM, w1t):
    def body(x_ref, deg_ref, w_ref, dinv_ref, y_ref):
        deg = deg_ref[0, :N, :] + deg_ref[1, :N, :] + 1.0
        dinv = lax.rsqrt(deg)
        dinv_ref[...] = dinv
        xw = jnp.dot(x_ref[...], w_ref[...], preferred_element_type=jnp.float32)
        y_ref[:N, :] = xw * dinv
        y_ref[N:, :] = jnp.zeros((NPY - N, H), jnp.float32)

    return pl.pallas_call(
        body,
        out_shape=(
            jax.ShapeDtypeStruct((N, H), jnp.float32),
            jax.ShapeDtypeStruct((NPY, H), jnp.float32),
        ),
    )(x, degM, w1t)


def _bn_relu(z, g_ref, be_ref):
    m = jnp.mean(z, axis=0, keepdims=True)
    zc = z - m
    v = jnp.mean(zc * zc, axis=0, keepdims=True)
    return zc * lax.rsqrt(v + EPS) * g_ref[...] + be_ref[...]


def _tc_mid(acc, y, dinvM, g, be, wnt):
    def body(acc_ref, y_ref, dinv_ref, g_ref, be_ref, w_ref, h_ref, yn_ref):
        z = (acc_ref[0, :N, :] + acc_ref[1, :N, :] + y_ref[:N, :]) * dinv_ref[...]
        h = jnp.maximum(_bn_relu(z, g_ref, be_ref), 0.0)
        h_ref[...] = h
        hw = jnp.dot(h, w_ref[...], preferred_element_type=jnp.float32)
        yn_ref[:N, :] = hw * dinv_ref[...]
        yn_ref[N:, :] = jnp.zeros((NPY - N, H), jnp.float32)

    return pl.pallas_call(
        body,
        out_shape=(
            jax.ShapeDtypeStruct((N, H), jnp.float32),
            jax.ShapeDtypeStruct((NPY, H), jnp.float32),
        ),
    )(acc, y, dinvM, g, be, wnt)


def _tc_res(acc, y, dinvM, g, be, res, wrest, bres, w4t):
    def body(acc_ref, y_ref, dinv_ref, g_ref, be_ref, res_ref, wr_ref, br_ref,
             w_ref, yn_ref):
        z = (acc_ref[0, :N, :] + acc_ref[1, :N, :] + y_ref[:N, :]) * dinv_ref[...]
        bn = _bn_relu(z, g_ref, be_ref)
        rw = jnp.dot(res_ref[...], wr_ref[...], preferred_element_type=jnp.float32)
        h = jnp.maximum(bn + rw + br_ref[...], 0.0)
        hw = jnp.dot(h, w_ref[...], preferred_element_type=jnp.float32)
        yn_ref[:N, :] = hw * dinv_ref[...]
        yn_ref[N:, :] = jnp.zeros((NPY - N, H), jnp.float32)

    return pl.pallas_call(
        body,
        out_shape=jax.ShapeDtypeStruct((NPY, H), jnp.float32),
    )(acc, y, dinvM, g, be, res, wrest, bres, w4t)


def _tc_head(acc, y, dinvM, g, be, wvt, bv, wot, bo, woutt, bout):
    def body(acc_ref, y_ref, dinv_ref, g_ref, be_ref, wv_ref, bv_ref, wo_ref,
             bo_ref, wout_ref, bout_ref, out_ref):
        z = (acc_ref[0, :N, :] + acc_ref[1, :N, :] + y_ref[:N, :]) * dinv_ref[...]
        h = jnp.maximum(_bn_relu(z, g_ref, be_ref), 0.0)
        v = jnp.dot(h, wv_ref[...], preferred_element_type=jnp.float32) + bv_ref[...]
        o = jnp.dot(v, wo_ref[...], preferred_element_type=jnp.float32) + bo_ref[...]
        out_ref[...] = (
            jnp.dot(o, wout_ref[...], preferred_element_type=jnp.float32)
            + bout_ref[...]
        )

    return pl.pallas_call(
        body,
        out_shape=jax.ShapeDtypeStruct((N, H), jnp.float32),
    )(acc, y, dinvM, g, be, wvt, bv, wot, bo, woutt, bout)


def kernel(x, edge_index, params):
    p = params
    pad = jnp.full((EPAD - E,), N, jnp.int32)
    srcf = jnp.concatenate([edge_index[0], pad]).reshape(NW, NSTEP, CH)
    dstf = jnp.concatenate([edge_index[1], pad]).reshape(NW, NSTEP, CH)
    sd = jnp.stack([srcf, dstf], axis=2)
    zeros_blk = jnp.zeros((SLC, H), jnp.float32)
    ones_blk = jnp.ones((CH, H), jnp.float32)

    def row(b):
        return b.reshape(1, H)

    degM = _sc_degree(sd, ones_blk, zeros_blk)
    dinvM, y1 = _tc_first(x, degM, p["W1"].T)
    acc1 = _sc_gather_scatter(y1, sd, zeros_blk)
    h1, y2 = _tc_mid(acc1, y1, dinvM, row(p["g1"]), row(p["be1"]), p["W2"].T)
    acc2 = _sc_gather_scatter(y2, sd, zeros_blk)
    _, y3 = _tc_mid(acc2, y2, dinvM, row(p["g2"]), row(p["be2"]), p["W3"].T)
    acc3 = _sc_gather_scatter(y3, sd, zeros_blk)
    y4 = _tc_res(acc3, y3, dinvM, row(p["g3"]), row(p["be3"]), h1,
                 p["Wres"].T, row(p["bres"]), p["W4"].T)
    acc4 = _sc_gather_scatter(y4, sd, zeros_blk)
    out = _tc_head(acc4, y4, dinvM, row(p["g4"]), row(p["be4"]),
                   p["Wv"].T, row(p["bv"]), p["Wo"].T, row(p["bo"]),
                   p["Wout"].T, row(p["bout"]))
    return out[None]


# speedup vs baseline: 1.6603x; 1.6603x over previous
import functools

import jax
import jax.numpy as jnp
from jax import lax
from jax.experimental import pallas as pl
from jax.experimental.pallas import tpu as pltpu
from jax.experimental.pallas import tpu_sc as plsc

N = 10000
H = 128
E = 320000
EPS = 1e-5
NC = 2
NS = 16
NW = NC * NS
CH = 128
NSTEP = 80
EPAD = NW * CH * NSTEP
NSTEP0 = 114
NSTEP1 = 44
EPAD2 = NS * CH * (NSTEP0 + NSTEP1)
NP = 10112
SLC = NP // NS
NPY = N + 16

_MESH = plsc.VectorSubcoreMesh(
    core_axis_name="c", subcore_axis_name="s", num_cores=NC, num_subcores=NS
)


def _sc_gather_scatter(y_pad, sd_idx, zeros_blk):
    @functools.partial(
        pl.kernel,
        out_type=jax.ShapeDtypeStruct((NC, NP, H), jnp.float32),
        mesh=_MESH,
        scratch_types=[
            pltpu.VMEM((2, CH), jnp.int32),
            pltpu.VMEM((2, CH), jnp.int32),
            pltpu.VMEM((CH, H), jnp.float32),
            pltpu.VMEM((CH, H), jnp.float32),
            pltpu.VMEM_SHARED((NP, H), jnp.float32),
            pltpu.SemaphoreType.DMA,
            pltpu.SemaphoreType.DMA,
            pltpu.SemaphoreType.DMA,
            pltpu.SemaphoreType.DMA,
            pltpu.SemaphoreType.DMA,
            pltpu.SemaphoreType.DMA,
        ],
    )
    def k(y_hbm, sd_hbm, z_hbm, acc_hbm, iA, iB, b0, b1, acc_sh,
          ia, ib, g0, g1, s0, s1):
        c = lax.axis_index("c")
        s = lax.axis_index("s")
        w = c * NS + s
        pltpu.sync_copy(z_hbm, acc_sh.at[pl.ds(s * SLC, SLC)])

        def idx_load(i, buf, sem):
            pltpu.async_copy(sd_hbm.at[w, i], buf, sem)

        def wait_idx(buf, sem):
            pltpu.make_async_copy(sd_hbm.at[0, 0], buf, sem).wait()

        def gather(buf_i, buf, sem):
            pltpu.async_copy(y_hbm.at[buf_i.at[0]], buf, sem)

        def wait_gather(buf, sem):
            pltpu.make_async_copy(y_hbm.at[pl.ds(0, CH)], buf, sem).wait()

        def scatter(buf_i, buf, sem):
            pltpu.async_copy(buf, acc_sh.at[buf_i.at[1]], sem, add=True)

        def wait_scatter(buf, sem):
            pltpu.make_async_copy(buf, acc_sh.at[pl.ds(0, CH)], sem).wait()

        nstep = jnp.where(c == 0, NSTEP0, NSTEP1)
        idx_load(0, iA, ia)
        idx_load(1, iB, ib)
        plsc.subcore_barrier()
        wait_idx(iA, ia)
        gather(iA, b0, g0)
        wait_idx(iB, ib)
        gather(iB, b1, g1)

        @pl.loop(0, NSTEP0, step=2)
        def _(i):
            @pl.when(i < nstep)
            def _():
                wait_gather(b0, g0)
                scatter(iA, b0, s0)
                wait_gather(b1, g1)
                scatter(iB, b1, s1)

                @pl.when(i + 2 < nstep)
                def _():
                    wait_scatter(b0, s0)
                    idx_load(i + 2, iA, ia)
                    wait_scatter(b1, s1)
                    idx_load(i + 3, iB, ib)
                    wait_idx(iA, ia)
                    gather(iA, b0, g0)
                    wait_idx(iB, ib)
                    gather(iB, b1, g1)

        wait_scatter(b0, s0)
        wait_scatter(b1, s1)
        plsc.subcore_barrier()
        pltpu.sync_copy(
            acc_sh.at[pl.ds(s * SLC, SLC)], acc_hbm.at[c, pl.ds(s * SLC, SLC)]
        )

    return k(y_pad, sd_idx, zeros_blk)


def _sc_degree(sd_idx, ones_blk, zeros_blk):
    @functools.partial(
        pl.kernel,
        out_type=jax.ShapeDtypeStruct((NC, NP, H), jnp.float32),
        mesh=_MESH,
        scratch_types=[
            pltpu.VMEM((2, CH), jnp.int32),
            pltpu.VMEM((2, CH), jnp.int32),
            pltpu.VMEM((CH, H), jnp.float32),
            pltpu.VMEM_SHARED((NP, H), jnp.float32),
            pltpu.SemaphoreType.DMA,
            pltpu.SemaphoreType.DMA,
            pltpu.SemaphoreType.DMA,
            pltpu.SemaphoreType.DMA,
        ],
    )
    def k(sd_hbm, o_hbm, z_hbm, acc_hbm, iA, iB, rows, acc_sh, ia, ib, s0, s1):
        c = lax.axis_index("c")
        s = lax.axis_index("s")
        w = c * NS + s
        pltpu.sync_copy(o_hbm, rows)
        pltpu.sync_copy(z_hbm, acc_sh.at[pl.ds(s * SLC, SLC)])

        def idx_load(i, buf, sem):
            pltpu.async_copy(sd_hbm.at[w, i], buf, sem)

        def wait_idx(buf, sem):
            pltpu.make_async_copy(sd_hbm.at[0, 0], buf, sem).wait()

        def scatter(buf_i, sem):
            pltpu.async_copy(rows, acc_sh.at[buf_i.at[1]], sem, add=True)

        def wait_scatter(sem):
            pltpu.make_async_copy(rows, acc_sh.at[pl.ds(0, CH)], sem).wait()

        nstep = jnp.where(c == 0, NSTEP0, NSTEP1)
        idx_load(0, iA, ia)
        idx_load(1, iB, ib)
        plsc.subcore_barrier()
        wait_idx(iA, ia)
        scatter(iA, s0)
        wait_idx(iB, ib)
        scatter(iB, s1)

        @pl.loop(0, NSTEP0, step=2)
        def _(i):
            @pl.when(i + 2 < nstep)
            def _():
                wait_scatter(s0)
                idx_load(i + 2, iA, ia)
                wait_scatter(s1)
                idx_load(i + 3, iB, ib)
                wait_idx(iA, ia)
                scatter(iA, s0)
                wait_idx(iB, ib)
                scatter(iB, s1)

        wait_scatter(s0)
        wait_scatter(s1)
        plsc.subcore_barrier()
        pltpu.sync_copy(
            acc_sh.at[pl.ds(s * SLC, SLC)], acc_hbm.at[c, pl.ds(s * SLC, SLC)]
        )

    return k(sd_idx, ones_blk, zeros_blk)


def _tc_first(x, degM, w1t):
    def body(x_ref, deg_ref, w_ref, dinv_ref, y_ref):
        deg = deg_ref[0, :N, :] + deg_ref[1, :N, :] + 1.0
        dinv = lax.rsqrt(deg)
        dinv_ref[...] = dinv
        xw = jnp.dot(x_ref[...], w_ref[...], preferred_element_type=jnp.float32)
        y_ref[:N, :] = xw * dinv
        y_ref[N:, :] = jnp.zeros((NPY - N, H), jnp.float32)

    return pl.pallas_call(
        body,
        out_shape=(
            jax.ShapeDtypeStruct((N, H), jnp.float32),
            jax.ShapeDtypeStruct((NPY, H), jnp.float32),
        ),
    )(x, degM, w1t)


def _bn_relu(z, g_ref, be_ref):
    m = jnp.mean(z, axis=0, keepdims=True)
    zc = z - m
    v = jnp.mean(zc * zc, axis=0, keepdims=True)
    return zc * lax.rsqrt(v + EPS) * g_ref[...] + be_ref[...]


def _tc_mid(acc, y, dinvM, g, be, wnt):
    def body(acc_ref, y_ref, dinv_ref, g_ref, be_ref, w_ref, h_ref, yn_ref):
        z = (acc_ref[0, :N, :] + acc_ref[1, :N, :] + y_ref[:N, :]) * dinv_ref[...]
        h = jnp.maximum(_bn_relu(z, g_ref, be_ref), 0.0)
        h_ref[...] = h
        hw = jnp.dot(h, w_ref[...], preferred_element_type=jnp.float32)
        yn_ref[:N, :] = hw * dinv_ref[...]
        yn_ref[N:, :] = jnp.zeros((NPY - N, H), jnp.float32)

    return pl.pallas_call(
        body,
        out_shape=(
            jax.ShapeDtypeStruct((N, H), jnp.float32),
            jax.ShapeDtypeStruct((NPY, H), jnp.float32),
        ),
    )(acc, y, dinvM, g, be, wnt)


def _tc_res(acc, y, dinvM, g, be, res, wrest, bres, w4t):
    def body(acc_ref, y_ref, dinv_ref, g_ref, be_ref, res_ref, wr_ref, br_ref,
             w_ref, yn_ref):
        z = (acc_ref[0, :N, :] + acc_ref[1, :N, :] + y_ref[:N, :]) * dinv_ref[...]
        bn = _bn_relu(z, g_ref, be_ref)
        rw = jnp.dot(res_ref[...], wr_ref[...], preferred_element_type=jnp.float32)
        h = jnp.maximum(bn + rw + br_ref[...], 0.0)
        hw = jnp.dot(h, w_ref[...], preferred_element_type=jnp.float32)
        yn_ref[:N, :] = hw * dinv_ref[...]
        yn_ref[N:, :] = jnp.zeros((NPY - N, H), jnp.float32)

    return pl.pallas_call(
        body,
        out_shape=jax.ShapeDtypeStruct((NPY, H), jnp.float32),
    )(acc, y, dinvM, g, be, res, wrest, bres, w4t)


def _tc_head(acc, y, dinvM, g, be, wvt, bv, wot, bo, woutt, bout):
    def body(acc_ref, y_ref, dinv_ref, g_ref, be_ref, wv_ref, bv_ref, wo_ref,
             bo_ref, wout_ref, bout_ref, out_ref):
        z = (acc_ref[0, :N, :] + acc_ref[1, :N, :] + y_ref[:N, :]) * dinv_ref[...]
        h = jnp.maximum(_bn_relu(z, g_ref, be_ref), 0.0)
        v = jnp.dot(h, wv_ref[...], preferred_element_type=jnp.float32) + bv_ref[...]
        o = jnp.dot(v, wo_ref[...], preferred_element_type=jnp.float32) + bo_ref[...]
        out_ref[...] = (
            jnp.dot(o, wout_ref[...], preferred_element_type=jnp.float32)
            + bout_ref[...]
        )

    return pl.pallas_call(
        body,
        out_shape=jax.ShapeDtypeStruct((N, H), jnp.float32),
    )(acc, y, dinvM, g, be, wvt, bv, wot, bo, woutt, bout)


def kernel(x, edge_index, params):
    p = params
    cnt = [NSTEP0 * CH] * NS + [NSTEP1 * CH] * NS
    offs = [0]
    for cn in cnt:
        offs.append(offs[-1] + cn)

    def pack(v):
        vp = jnp.concatenate([v, jnp.full((EPAD2 - E,), N, jnp.int32)])
        rows = []
        for w in range(NW):
            sl = vp[offs[w]:offs[w + 1]]
            if cnt[w] < NSTEP0 * CH:
                sl = jnp.concatenate(
                    [sl, jnp.full((NSTEP0 * CH - cnt[w],), N, jnp.int32)]
                )
            rows.append(sl.reshape(NSTEP0, CH))
        return jnp.stack(rows)

    sd = jnp.stack([pack(edge_index[0]), pack(edge_index[1])], axis=2)
    zeros_blk = jnp.zeros((SLC, H), jnp.float32)
    ones_blk = jnp.ones((CH, H), jnp.float32)

    def row(b):
        return b.reshape(1, H)

    degM = _sc_degree(sd, ones_blk, zeros_blk)
    dinvM, y1 = _tc_first(x, degM, p["W1"].T)
    acc1 = _sc_gather_scatter(y1, sd, zeros_blk)
    h1, y2 = _tc_mid(acc1, y1, dinvM, row(p["g1"]), row(p["be1"]), p["W2"].T)
    acc2 = _sc_gather_scatter(y2, sd, zeros_blk)
    _, y3 = _tc_mid(acc2, y2, dinvM, row(p["g2"]), row(p["be2"]), p["W3"].T)
    acc3 = _sc_gather_scatter(y3, sd, zeros_blk)
    y4 = _tc_res(acc3, y3, dinvM, row(p["g3"]), row(p["be3"]), h1,
                 p["Wres"].T, row(p["bres"]), p["W4"].T)
    acc4 = _sc_gather_scatter(y4, sd, zeros_blk)
    out = _tc_head(acc4, y4, dinvM, row(p["g4"]), row(p["be4"]),
                   p["Wv"].T, row(p["bv"]), p["Wo"].T, row(p["bo"]),
                   p["Wout"].T, row(p["bout"]))
    return out[None]


# speedup vs baseline: 1.7243x; 1.0385x over previous
import functools

import jax
import jax.numpy as jnp
from jax import lax
from jax.experimental import pallas as pl
from jax.experimental.pallas import tpu as pltpu
from jax.experimental.pallas import tpu_sc as plsc

N = 10000
H = 128
E = 320000
EPS = 1e-5
NC = 2
NS = 16
NW = NC * NS
CH = 128
NSTEP = 80
EPAD = NW * CH * NSTEP
NSTEP0 = 118
NSTEP1 = 40
EPAD2 = NS * CH * (NSTEP0 + NSTEP1)
NP = 10112
SLC = NP // NS
NPY = N + 16

_MESH = plsc.VectorSubcoreMesh(
    core_axis_name="c", subcore_axis_name="s", num_cores=NC, num_subcores=NS
)


def _sc_gather_scatter(y_pad, sd_idx, zeros_blk):
    @functools.partial(
        pl.kernel,
        out_type=jax.ShapeDtypeStruct((NC, NP, H), jnp.float32),
        mesh=_MESH,
        scratch_types=[
            pltpu.VMEM((2, CH), jnp.int32),
            pltpu.VMEM((2, CH), jnp.int32),
            pltpu.VMEM((CH, H), jnp.float32),
            pltpu.VMEM((CH, H), jnp.float32),
            pltpu.VMEM_SHARED((NP, H), jnp.float32),
            pltpu.SemaphoreType.DMA,
            pltpu.SemaphoreType.DMA,
            pltpu.SemaphoreType.DMA,
            pltpu.SemaphoreType.DMA,
            pltpu.SemaphoreType.DMA,
            pltpu.SemaphoreType.DMA,
        ],
    )
    def k(y_hbm, sd_hbm, z_hbm, acc_hbm, iA, iB, b0, b1, acc_sh,
          ia, ib, g0, g1, s0, s1):
        c = lax.axis_index("c")
        s = lax.axis_index("s")
        w = c * NS + s
        pltpu.sync_copy(z_hbm, acc_sh.at[pl.ds(s * SLC, SLC)])

        def idx_load(i, buf, sem):
            pltpu.async_copy(sd_hbm.at[w, i], buf, sem)

        def wait_idx(buf, sem):
            pltpu.make_async_copy(sd_hbm.at[0, 0], buf, sem).wait()

        def gather(buf_i, buf, sem):
            pltpu.async_copy(y_hbm.at[buf_i.at[0]], buf, sem)

        def wait_gather(buf, sem):
            pltpu.make_async_copy(y_hbm.at[pl.ds(0, CH)], buf, sem).wait()

        def scatter(buf_i, buf, sem):
            pltpu.async_copy(buf, acc_sh.at[buf_i.at[1]], sem, add=True)

        def wait_scatter(buf, sem):
            pltpu.make_async_copy(buf, acc_sh.at[pl.ds(0, CH)], sem).wait()

        nstep = jnp.where(c == 0, NSTEP0, NSTEP1)
        idx_load(0, iA, ia)
        idx_load(1, iB, ib)
        plsc.subcore_barrier()
        wait_idx(iA, ia)
        gather(iA, b0, g0)
        wait_idx(iB, ib)
        gather(iB, b1, g1)

        @pl.loop(0, NSTEP0, step=2)
        def _(i):
            @pl.when(i < nstep)
            def _():
                wait_gather(b0, g0)
                scatter(iA, b0, s0)
                wait_gather(b1, g1)
                scatter(iB, b1, s1)

                @pl.when(i + 2 < nstep)
                def _():
                    wait_scatter(b0, s0)
                    idx_load(i + 2, iA, ia)
                    wait_scatter(b1, s1)
                    idx_load(i + 3, iB, ib)
                    wait_idx(iA, ia)
                    gather(iA, b0, g0)
                    wait_idx(iB, ib)
                    gather(iB, b1, g1)

        wait_scatter(b0, s0)
        wait_scatter(b1, s1)
        plsc.subcore_barrier()
        pltpu.sync_copy(
            acc_sh.at[pl.ds(s * SLC, SLC)], acc_hbm.at[c, pl.ds(s * SLC, SLC)]
        )

    return k(y_pad, sd_idx, zeros_blk)


def _sc_degree(sd_idx, ones_blk, zeros_blk):
    @functools.partial(
        pl.kernel,
        out_type=jax.ShapeDtypeStruct((NC, NP, H), jnp.float32),
        mesh=_MESH,
        scratch_types=[
            pltpu.VMEM((2, CH), jnp.int32),
            pltpu.VMEM((2, CH), jnp.int32),
            pltpu.VMEM((CH, H), jnp.float32),
            pltpu.VMEM_SHARED((NP, H), jnp.float32),
            pltpu.SemaphoreType.DMA,
            pltpu.SemaphoreType.DMA,
            pltpu.SemaphoreType.DMA,
            pltpu.SemaphoreType.DMA,
        ],
    )
    def k(sd_hbm, o_hbm, z_hbm, acc_hbm, iA, iB, rows, acc_sh, ia, ib, s0, s1):
        c = lax.axis_index("c")
        s = lax.axis_index("s")
        w = c * NS + s
        pltpu.sync_copy(o_hbm, rows)
        pltpu.sync_copy(z_hbm, acc_sh.at[pl.ds(s * SLC, SLC)])

        def idx_load(i, buf, sem):
            pltpu.async_copy(sd_hbm.at[w, i], buf, sem)

        def wait_idx(buf, sem):
            pltpu.make_async_copy(sd_hbm.at[0, 0], buf, sem).wait()

        def scatter(buf_i, sem):
            pltpu.async_copy(rows, acc_sh.at[buf_i.at[1]], sem, add=True)

        def wait_scatter(sem):
            pltpu.make_async_copy(rows, acc_sh.at[pl.ds(0, CH)], sem).wait()

        nstep = jnp.where(c == 0, NSTEP0, NSTEP1)
        idx_load(0, iA, ia)
        idx_load(1, iB, ib)
        plsc.subcore_barrier()
        wait_idx(iA, ia)
        scatter(iA, s0)
        wait_idx(iB, ib)
        scatter(iB, s1)

        @pl.loop(0, NSTEP0, step=2)
        def _(i):
            @pl.when(i + 2 < nstep)
            def _():
                wait_scatter(s0)
                idx_load(i + 2, iA, ia)
                wait_scatter(s1)
                idx_load(i + 3, iB, ib)
                wait_idx(iA, ia)
                scatter(iA, s0)
                wait_idx(iB, ib)
                scatter(iB, s1)

        wait_scatter(s0)
        wait_scatter(s1)
        plsc.subcore_barrier()
        pltpu.sync_copy(
            acc_sh.at[pl.ds(s * SLC, SLC)], acc_hbm.at[c, pl.ds(s * SLC, SLC)]
        )

    return k(sd_idx, ones_blk, zeros_blk)


def _tc_first(x, degM, w1t):
    def body(x_ref, deg_ref, w_ref, dinv_ref, y_ref):
        deg = deg_ref[0, :N, :] + deg_ref[1, :N, :] + 1.0
        dinv = lax.rsqrt(deg)
        dinv_ref[...] = dinv
        xw = jnp.dot(x_ref[...], w_ref[...], preferred_element_type=jnp.float32)
        y_ref[:N, :] = xw * dinv
        y_ref[N:, :] = jnp.zeros((NPY - N, H), jnp.float32)

    return pl.pallas_call(
        body,
        out_shape=(
            jax.ShapeDtypeStruct((N, H), jnp.float32),
            jax.ShapeDtypeStruct((NPY, H), jnp.float32),
        ),
    )(x, degM, w1t)


def _bn_relu(z, g_ref, be_ref):
    m = jnp.mean(z, axis=0, keepdims=True)
    zc = z - m
    v = jnp.mean(zc * zc, axis=0, keepdims=True)
    return zc * lax.rsqrt(v + EPS) * g_ref[...] + be_ref[...]


def _tc_mid(acc, y, dinvM, g, be, wnt):
    def body(acc_ref, y_ref, dinv_ref, g_ref, be_ref, w_ref, h_ref, yn_ref):
        z = (acc_ref[0, :N, :] + acc_ref[1, :N, :] + y_ref[:N, :]) * dinv_ref[...]
        h = jnp.maximum(_bn_relu(z, g_ref, be_ref), 0.0)
        h_ref[...] = h
        hw = jnp.dot(h, w_ref[...], preferred_element_type=jnp.float32)
        yn_ref[:N, :] = hw * dinv_ref[...]
        yn_ref[N:, :] = jnp.zeros((NPY - N, H), jnp.float32)

    return pl.pallas_call(
        body,
        out_shape=(
            jax.ShapeDtypeStruct((N, H), jnp.float32),
            jax.ShapeDtypeStruct((NPY, H), jnp.float32),
        ),
    )(acc, y, dinvM, g, be, wnt)


def _tc_res(acc, y, dinvM, g, be, res, wrest, bres, w4t):
    def body(acc_ref, y_ref, dinv_ref, g_ref, be_ref, res_ref, wr_ref, br_ref,
             w_ref, yn_ref):
        z = (acc_ref[0, :N, :] + acc_ref[1, :N, :] + y_ref[:N, :]) * dinv_ref[...]
        bn = _bn_relu(z, g_ref, be_ref)
        rw = jnp.dot(res_ref[...], wr_ref[...], preferred_element_type=jnp.float32)
        h = jnp.maximum(bn + rw + br_ref[...], 0.0)
        hw = jnp.dot(h, w_ref[...], preferred_element_type=jnp.float32)
        yn_ref[:N, :] = hw * dinv_ref[...]
        yn_ref[N:, :] = jnp.zeros((NPY - N, H), jnp.float32)

    return pl.pallas_call(
        body,
        out_shape=jax.ShapeDtypeStruct((NPY, H), jnp.float32),
    )(acc, y, dinvM, g, be, res, wrest, bres, w4t)


def _tc_head(acc, y, dinvM, g, be, wvt, bv, wot, bo, woutt, bout):
    def body(acc_ref, y_ref, dinv_ref, g_ref, be_ref, wv_ref, bv_ref, wo_ref,
             bo_ref, wout_ref, bout_ref, out_ref):
        z = (acc_ref[0, :N, :] + acc_ref[1, :N, :] + y_ref[:N, :]) * dinv_ref[...]
        h = jnp.maximum(_bn_relu(z, g_ref, be_ref), 0.0)
        v = jnp.dot(h, wv_ref[...], preferred_element_type=jnp.float32) + bv_ref[...]
        o = jnp.dot(v, wo_ref[...], preferred_element_type=jnp.float32) + bo_ref[...]
        out_ref[...] = (
            jnp.dot(o, wout_ref[...], preferred_element_type=jnp.float32)
            + bout_ref[...]
        )

    return pl.pallas_call(
        body,
        out_shape=jax.ShapeDtypeStruct((N, H), jnp.float32),
    )(acc, y, dinvM, g, be, wvt, bv, wot, bo, woutt, bout)


def kernel(x, edge_index, params):
    p = params
    cnt = [NSTEP0 * CH] * NS + [NSTEP1 * CH] * NS
    offs = [0]
    for cn in cnt:
        offs.append(offs[-1] + cn)

    def pack(v):
        vp = jnp.concatenate([v, jnp.full((EPAD2 - E,), N, jnp.int32)])
        rows = []
        for w in range(NW):
            sl = vp[offs[w]:offs[w + 1]]
            if cnt[w] < NSTEP0 * CH:
                sl = jnp.concatenate(
                    [sl, jnp.full((NSTEP0 * CH - cnt[w],), N, jnp.int32)]
                )
            rows.append(sl.reshape(NSTEP0, CH))
        return jnp.stack(rows)

    sd = jnp.stack([pack(edge_index[0]), pack(edge_index[1])], axis=2)
    zeros_blk = jnp.zeros((SLC, H), jnp.float32)
    ones_blk = jnp.ones((CH, H), jnp.float32)

    def row(b):
        return b.reshape(1, H)

    degM = _sc_degree(sd, ones_blk, zeros_blk)
    dinvM, y1 = _tc_first(x, degM, p["W1"].T)
    acc1 = _sc_gather_scatter(y1, sd, zeros_blk)
    h1, y2 = _tc_mid(acc1, y1, dinvM, row(p["g1"]), row(p["be1"]), p["W2"].T)
    acc2 = _sc_gather_scatter(y2, sd, zeros_blk)
    _, y3 = _tc_mid(acc2, y2, dinvM, row(p["g2"]), row(p["be2"]), p["W3"].T)
    acc3 = _sc_gather_scatter(y3, sd, zeros_blk)
    y4 = _tc_res(acc3, y3, dinvM, row(p["g3"]), row(p["be3"]), h1,
                 p["Wres"].T, row(p["bres"]), p["W4"].T)
    acc4 = _sc_gather_scatter(y4, sd, zeros_blk)
    out = _tc_head(acc4, y4, dinvM, row(p["g4"]), row(p["be4"]),
                   p["Wv"].T, row(p["bv"]), p["Wo"].T, row(p["bo"]),
                   p["Wout"].T, row(p["bout"]))
    return out[None]


# speedup vs baseline: 1.7695x; 1.0262x over previous
import functools

import jax
import jax.numpy as jnp
from jax import lax
from jax.experimental import pallas as pl
from jax.experimental.pallas import tpu as pltpu
from jax.experimental.pallas import tpu_sc as plsc

N = 10000
H = 128
E = 320000
EPS = 1e-5
NC = 2
NS = 16
NW = NC * NS
CH = 128
NSTEP = 80
EPAD = NW * CH * NSTEP
NSTEP0 = 118
NSTEP1 = 40
NCHUNK = NS * (NSTEP0 + NSTEP1)
C0TOT = NS * NSTEP0
EPAD2 = NCHUNK * CH
NSTEPD = NCHUNK // NW
NP = 10112
SLC = NP // NS
NPY = N + 16

_MESH = plsc.VectorSubcoreMesh(
    core_axis_name="c", subcore_axis_name="s", num_cores=NC, num_subcores=NS
)


def _sc_gather_scatter(y_pad, sd_idx, zeros_blk):
    @functools.partial(
        pl.kernel,
        out_type=jax.ShapeDtypeStruct((NC, NP, H), jnp.float32),
        mesh=_MESH,
        scratch_types=[
            pltpu.VMEM((2, CH), jnp.int32),
            pltpu.VMEM((2, CH), jnp.int32),
            pltpu.VMEM((CH, H), jnp.float32),
            pltpu.VMEM((CH, H), jnp.float32),
            pltpu.VMEM_SHARED((NP, H), jnp.float32),
            pltpu.SemaphoreType.DMA,
            pltpu.SemaphoreType.DMA,
            pltpu.SemaphoreType.DMA,
            pltpu.SemaphoreType.DMA,
            pltpu.SemaphoreType.DMA,
            pltpu.SemaphoreType.DMA,
        ],
    )
    def k(y_hbm, sd_hbm, z_hbm, acc_hbm, iA, iB, b0, b1, acc_sh,
          ia, ib, g0, g1, s0, s1):
        c = lax.axis_index("c")
        s = lax.axis_index("s")
        base = jnp.where(c == 0, s * NSTEP0, C0TOT + s * NSTEP1)
        pltpu.sync_copy(z_hbm, acc_sh.at[pl.ds(s * SLC, SLC)])

        def idx_load(i, buf, sem):
            pltpu.async_copy(sd_hbm.at[base + i], buf, sem)

        def wait_idx(buf, sem):
            pltpu.make_async_copy(sd_hbm.at[0], buf, sem).wait()

        def gather(buf_i, buf, sem):
            pltpu.async_copy(y_hbm.at[buf_i.at[0]], buf, sem)

        def wait_gather(buf, sem):
            pltpu.make_async_copy(y_hbm.at[pl.ds(0, CH)], buf, sem).wait()

        def scatter(buf_i, buf, sem):
            pltpu.async_copy(buf, acc_sh.at[buf_i.at[1]], sem, add=True)

        def wait_scatter(buf, sem):
            pltpu.make_async_copy(buf, acc_sh.at[pl.ds(0, CH)], sem).wait()

        nstep = jnp.where(c == 0, NSTEP0, NSTEP1)
        idx_load(0, iA, ia)
        idx_load(1, iB, ib)
        plsc.subcore_barrier()
        wait_idx(iA, ia)
        gather(iA, b0, g0)
        wait_idx(iB, ib)
        gather(iB, b1, g1)

        @pl.loop(0, NSTEP0, step=2)
        def _(i):
            @pl.when(i < nstep)
            def _():
                wait_gather(b0, g0)
                scatter(iA, b0, s0)
                wait_gather(b1, g1)
                scatter(iB, b1, s1)

                @pl.when(i + 2 < nstep)
                def _():
                    wait_scatter(b0, s0)
                    idx_load(i + 2, iA, ia)
                    wait_scatter(b1, s1)
                    idx_load(i + 3, iB, ib)
                    wait_idx(iA, ia)
                    gather(iA, b0, g0)
                    wait_idx(iB, ib)
                    gather(iB, b1, g1)

        wait_scatter(b0, s0)
        wait_scatter(b1, s1)
        plsc.subcore_barrier()
        pltpu.sync_copy(
            acc_sh.at[pl.ds(s * SLC, SLC)], acc_hbm.at[c, pl.ds(s * SLC, SLC)]
        )

    return k(y_pad, sd_idx, zeros_blk)


def _sc_degree(sd_idx, ones_blk, zeros_blk):
    @functools.partial(
        pl.kernel,
        out_type=jax.ShapeDtypeStruct((NC, NP, H), jnp.float32),
        mesh=_MESH,
        scratch_types=[
            pltpu.VMEM((2, CH), jnp.int32),
            pltpu.VMEM((2, CH), jnp.int32),
            pltpu.VMEM((CH, H), jnp.float32),
            pltpu.VMEM_SHARED((NP, H), jnp.float32),
            pltpu.SemaphoreType.DMA,
            pltpu.SemaphoreType.DMA,
            pltpu.SemaphoreType.DMA,
            pltpu.SemaphoreType.DMA,
        ],
    )
    def k(sd_hbm, o_hbm, z_hbm, acc_hbm, iA, iB, rows, acc_sh, ia, ib, s0, s1):
        c = lax.axis_index("c")
        s = lax.axis_index("s")
        base = (c * NS + s) * NSTEPD
        pltpu.sync_copy(o_hbm, rows)
        pltpu.sync_copy(z_hbm, acc_sh.at[pl.ds(s * SLC, SLC)])

        def idx_load(i, buf, sem):
            pltpu.async_copy(sd_hbm.at[base + i], buf, sem)

        def wait_idx(buf, sem):
            pltpu.make_async_copy(sd_hbm.at[0], buf, sem).wait()

        def scatter(buf_i, sem):
            pltpu.async_copy(rows, acc_sh.at[buf_i.at[1]], sem, add=True)

        def wait_scatter(sem):
            pltpu.make_async_copy(rows, acc_sh.at[pl.ds(0, CH)], sem).wait()

        idx_load(0, iA, ia)
        idx_load(1, iB, ib)
        plsc.subcore_barrier()
        wait_idx(iA, ia)
        scatter(iA, s0)
        wait_idx(iB, ib)
        scatter(iB, s1)

        @pl.loop(0, NSTEPD - 1, step=2)
        def _(i):
            @pl.when(i + 2 < NSTEPD)
            def _():
                wait_scatter(s0)
                idx_load(i + 2, iA, ia)
                wait_idx(iA, ia)
                scatter(iA, s0)

            @pl.when(i + 3 < NSTEPD)
            def _():
                wait_scatter(s1)
                idx_load(i + 3, iB, ib)
                wait_idx(iB, ib)
                scatter(iB, s1)

        wait_scatter(s0)
        wait_scatter(s1)
        plsc.subcore_barrier()
        pltpu.sync_copy(
            acc_sh.at[pl.ds(s * SLC, SLC)], acc_hbm.at[c, pl.ds(s * SLC, SLC)]
        )

    return k(sd_idx, ones_blk, zeros_blk)


def _tc_first(x, degM, w1t):
    def body(x_ref, deg_ref, w_ref, dinv_ref, y_ref):
        deg = deg_ref[0, :N, :] + deg_ref[1, :N, :] + 1.0
        dinv = lax.rsqrt(deg)
        dinv_ref[...] = dinv
        xw = jnp.dot(x_ref[...], w_ref[...], preferred_element_type=jnp.float32)
        y_ref[:N, :] = xw * dinv
        y_ref[N:, :] = jnp.zeros((NPY - N, H), jnp.float32)

    return pl.pallas_call(
        body,
        out_shape=(
            jax.ShapeDtypeStruct((N, H), jnp.float32),
            jax.ShapeDtypeStruct((NPY, H), jnp.float32),
        ),
    )(x, degM, w1t)


def _bn_relu(z, g_ref, be_ref):
    m = jnp.mean(z, axis=0, keepdims=True)
    zc = z - m
    v = jnp.mean(zc * zc, axis=0, keepdims=True)
    return zc * lax.rsqrt(v + EPS) * g_ref[...] + be_ref[...]


def _tc_mid(acc, y, dinvM, g, be, wnt):
    def body(acc_ref, y_ref, dinv_ref, g_ref, be_ref, w_ref, h_ref, yn_ref):
        z = (acc_ref[0, :N, :] + acc_ref[1, :N, :] + y_ref[:N, :]) * dinv_ref[...]
        h = jnp.maximum(_bn_relu(z, g_ref, be_ref), 0.0)
        h_ref[...] = h
        hw = jnp.dot(h, w_ref[...], preferred_element_type=jnp.float32)
        yn_ref[:N, :] = hw * dinv_ref[...]
        yn_ref[N:, :] = jnp.zeros((NPY - N, H), jnp.float32)

    return pl.pallas_call(
        body,
        out_shape=(
            jax.ShapeDtypeStruct((N, H), jnp.float32),
            jax.ShapeDtypeStruct((NPY, H), jnp.float32),
        ),
    )(acc, y, dinvM, g, be, wnt)


def _tc_res(acc, y, dinvM, g, be, res, wrest, bres, w4t):
    def body(acc_ref, y_ref, dinv_ref, g_ref, be_ref, res_ref, wr_ref, br_ref,
             w_ref, yn_ref):
        z = (acc_ref[0, :N, :] + acc_ref[1, :N, :] + y_ref[:N, :]) * dinv_ref[...]
        bn = _bn_relu(z, g_ref, be_ref)
        rw = jnp.dot(res_ref[...], wr_ref[...], preferred_element_type=jnp.float32)
        h = jnp.maximum(bn + rw + br_ref[...], 0.0)
        hw = jnp.dot(h, w_ref[...], preferred_element_type=jnp.float32)
        yn_ref[:N, :] = hw * dinv_ref[...]
        yn_ref[N:, :] = jnp.zeros((NPY - N, H), jnp.float32)

    return pl.pallas_call(
        body,
        out_shape=jax.ShapeDtypeStruct((NPY, H), jnp.float32),
    )(acc, y, dinvM, g, be, res, wrest, bres, w4t)


def _tc_head(acc, y, dinvM, g, be, wvt, bv, wot, bo, woutt, bout):
    def body(acc_ref, y_ref, dinv_ref, g_ref, be_ref, wv_ref, bv_ref, wo_ref,
             bo_ref, wout_ref, bout_ref, out_ref):
        z = (acc_ref[0, :N, :] + acc_ref[1, :N, :] + y_ref[:N, :]) * dinv_ref[...]
        h = jnp.maximum(_bn_relu(z, g_ref, be_ref), 0.0)
        v = jnp.dot(h, wv_ref[...], preferred_element_type=jnp.float32) + bv_ref[...]
        o = jnp.dot(v, wo_ref[...], preferred_element_type=jnp.float32) + bo_ref[...]
        out_ref[...] = (
            jnp.dot(o, wout_ref[...], preferred_element_type=jnp.float32)
            + bout_ref[...]
        )

    return pl.pallas_call(
        body,
        out_shape=jax.ShapeDtypeStruct((N, H), jnp.float32),
    )(acc, y, dinvM, g, be, wvt, bv, wot, bo, woutt, bout)


def kernel(x, edge_index, params):
    p = params
    pad = jnp.full((EPAD2 - E,), N, jnp.int32)
    srcf = jnp.concatenate([edge_index[0], pad]).reshape(NCHUNK, CH)
    dstf = jnp.concatenate([edge_index[1], pad]).reshape(NCHUNK, CH)
    sd = jnp.stack([srcf, dstf], axis=1)
    zeros_blk = jnp.zeros((SLC, H), jnp.float32)
    ones_blk = jnp.ones((CH, H), jnp.float32)

    def row(b):
        return b.reshape(1, H)

    degM = _sc_degree(sd, ones_blk, zeros_blk)
    dinvM, y1 = _tc_first(x, degM, p["W1"].T)
    acc1 = _sc_gather_scatter(y1, sd, zeros_blk)
    h1, y2 = _tc_mid(acc1, y1, dinvM, row(p["g1"]), row(p["be1"]), p["W2"].T)
    acc2 = _sc_gather_scatter(y2, sd, zeros_blk)
    _, y3 = _tc_mid(acc2, y2, dinvM, row(p["g2"]), row(p["be2"]), p["W3"].T)
    acc3 = _sc_gather_scatter(y3, sd, zeros_blk)
    y4 = _tc_res(acc3, y3, dinvM, row(p["g3"]), row(p["be3"]), h1,
                 p["Wres"].T, row(p["bres"]), p["W4"].T)
    acc4 = _sc_gather_scatter(y4, sd, zeros_blk)
    out = _tc_head(acc4, y4, dinvM, row(p["g4"]), row(p["be4"]),
                   p["Wv"].T, row(p["bv"]), p["Wo"].T, row(p["bo"]),
                   p["Wout"].T, row(p["bout"]))
    return out[None]


# speedup vs baseline: 1.8354x; 1.0373x over previous
import dataclasses
import functools

import jax
import jax.numpy as jnp
from jax import lax
from jax.experimental import pallas as pl
from jax.experimental.pallas import tpu as pltpu
from jax.experimental.pallas import tpu_sc as plsc

N = 10000
H = 128
E = 320000
EPS = 1e-5
NC = 2
NS = 16
NW = NC * NS
CH = 128
NSTEP = 80
EPAD = NW * CH * NSTEP
NSTEP0 = 118
NSTEP1 = 40
NCHUNK = NS * (NSTEP0 + NSTEP1)
C0TOT = NS * NSTEP0
EPAD2 = NCHUNK * CH
NSTEPD = NCHUNK // NW
NP = 10112
SLC = NP // NS
NPY = N + 16
NR = 80

_MESH = plsc.VectorSubcoreMesh(
    core_axis_name="c", subcore_axis_name="s", num_cores=NC, num_subcores=NS
)

_CP = pltpu.CompilerParams()
if "needs_layout_passes" in pltpu.CompilerParams.__dataclass_fields__:
    _CP = dataclasses.replace(_CP, needs_layout_passes=False)


def _sc_gather_scatter(y_pad, sd_idx, zeros_blk):
    @functools.partial(
        pl.kernel,
        out_type=jax.ShapeDtypeStruct((NC, NP, H), jnp.float32),
        mesh=_MESH,
        scratch_types=[
            pltpu.VMEM((2, CH), jnp.int32),
            pltpu.VMEM((2, CH), jnp.int32),
            pltpu.VMEM((CH, H), jnp.float32),
            pltpu.VMEM((CH, H), jnp.float32),
            pltpu.VMEM_SHARED((NP, H), jnp.float32),
            pltpu.SemaphoreType.DMA,
            pltpu.SemaphoreType.DMA,
            pltpu.SemaphoreType.DMA,
            pltpu.SemaphoreType.DMA,
            pltpu.SemaphoreType.DMA,
            pltpu.SemaphoreType.DMA,
        ],
    )
    def k(y_hbm, sd_hbm, z_hbm, acc_hbm, iA, iB, b0, b1, acc_sh,
          ia, ib, g0, g1, s0, s1):
        c = lax.axis_index("c")
        s = lax.axis_index("s")
        base = jnp.where(c == 0, s * NSTEP0, C0TOT + s * NSTEP1)
        pltpu.sync_copy(z_hbm, acc_sh.at[pl.ds(s * SLC, SLC)])

        def idx_load(i, buf, sem):
            pltpu.async_copy(sd_hbm.at[base + i], buf, sem)

        def wait_idx(buf, sem):
            pltpu.make_async_copy(sd_hbm.at[0], buf, sem).wait()

        def gather(buf_i, buf, sem):
            pltpu.async_copy(y_hbm.at[buf_i.at[0]], buf, sem)

        def wait_gather(buf, sem):
            pltpu.make_async_copy(y_hbm.at[pl.ds(0, CH)], buf, sem).wait()

        def scatter(buf_i, buf, sem):
            pltpu.async_copy(buf, acc_sh.at[buf_i.at[1]], sem, add=True)

        def wait_scatter(buf, sem):
            pltpu.make_async_copy(buf, acc_sh.at[pl.ds(0, CH)], sem).wait()

        nstep = jnp.where(c == 0, NSTEP0, NSTEP1)
        idx_load(0, iA, ia)
        idx_load(1, iB, ib)
        plsc.subcore_barrier()
        wait_idx(iA, ia)
        gather(iA, b0, g0)
        wait_idx(iB, ib)
        gather(iB, b1, g1)

        @pl.loop(0, NSTEP0, step=2)
        def _(i):
            @pl.when(i < nstep)
            def _():
                wait_gather(b0, g0)
                scatter(iA, b0, s0)
                wait_gather(b1, g1)
                scatter(iB, b1, s1)

                @pl.when(i + 2 < nstep)
                def _():
                    wait_scatter(b0, s0)
                    idx_load(i + 2, iA, ia)
                    wait_scatter(b1, s1)
                    idx_load(i + 3, iB, ib)
                    wait_idx(iA, ia)
                    gather(iA, b0, g0)
                    wait_idx(iB, ib)
                    gather(iB, b1, g1)

        wait_scatter(b0, s0)
        wait_scatter(b1, s1)
        plsc.subcore_barrier()
        pltpu.sync_copy(
            acc_sh.at[pl.ds(s * SLC, SLC)], acc_hbm.at[c, pl.ds(s * SLC, SLC)]
        )

    return k(y_pad, sd_idx, zeros_blk)


def _sc_degree(sd_idx, id_rows, zeros_blk):
    @functools.partial(
        pl.kernel,
        out_type=jax.ShapeDtypeStruct((NC, NR, H), jnp.float32),
        mesh=_MESH,
        compiler_params=_CP,
        scratch_types=[
            pltpu.VMEM((NSTEPD, 2, CH), jnp.int32),
            pltpu.VMEM((NR, H), jnp.float32),
            pltpu.VMEM((1, NR), jnp.int32),
            pltpu.VMEM_SHARED((NR, H), jnp.float32),
        ],
    )
    def k(sd_hbm, id_hbm, z_hbm, out_hbm, idx_all, hist, id_v, acc_sh):
        c = lax.axis_index("c")
        s = lax.axis_index("s")
        base = (c * NS + s) * NSTEPD
        pltpu.sync_copy(sd_hbm.at[pl.ds(base, NSTEPD)], idx_all)
        pltpu.sync_copy(id_hbm, id_v)

        @pl.when(s == 0)
        def _():
            pltpu.sync_copy(z_hbm.at[pl.ds(0, NR)], acc_sh)


        @pl.loop(0, NR)
        def _(r):
            for g in range(8):
                hist[r, pl.ds(16 * g, 16)] = jnp.zeros((16,), jnp.float32)

        ones16 = jnp.ones((16,), jnp.float32)

        @pl.loop(0, NSTEPD)
        def _(i):
            for g in range(8):
                d = idx_all[i, 1, pl.ds(16 * g, 16)]
                plsc.addupdate_scatter(hist, [d >> 7, d & 127], ones16)

        plsc.subcore_barrier()
        pltpu.sync_copy(hist, acc_sh.at[id_v.at[0]], add=True)
        plsc.subcore_barrier()

        @pl.when(s == 0)
        def _():
            pltpu.sync_copy(acc_sh, out_hbm.at[c])

    return k(sd_idx, id_rows, zeros_blk)


def _tc_first(x, degM, w1t):
    def body(x_ref, deg_ref, w_ref, dinv_ref, y_ref):
        deg = deg_ref[0, :N, :] + deg_ref[1, :N, :] + 1.0
        dinv = lax.rsqrt(deg) * jnp.ones((1, H), jnp.float32)
        dinv_ref[...] = dinv
        xw = jnp.dot(x_ref[...], w_ref[...], preferred_element_type=jnp.float32)
        y_ref[:N, :] = xw * dinv
        y_ref[N:, :] = jnp.zeros((NPY - N, H), jnp.float32)

    return pl.pallas_call(
        body,
        out_shape=(
            jax.ShapeDtypeStruct((N, H), jnp.float32),
            jax.ShapeDtypeStruct((NPY, H), jnp.float32),
        ),
    )(x, degM, w1t)


def _bn_relu(z, g_ref, be_ref):
    m = jnp.mean(z, axis=0, keepdims=True)
    zc = z - m
    v = jnp.mean(zc * zc, axis=0, keepdims=True)
    return zc * lax.rsqrt(v + EPS) * g_ref[...] + be_ref[...]


def _tc_mid(acc, y, dinvM, g, be, wnt):
    def body(acc_ref, y_ref, dinv_ref, g_ref, be_ref, w_ref, h_ref, yn_ref):
        z = (acc_ref[0, :N, :] + acc_ref[1, :N, :] + y_ref[:N, :]) * dinv_ref[...]
        h = jnp.maximum(_bn_relu(z, g_ref, be_ref), 0.0)
        h_ref[...] = h
        hw = jnp.dot(h, w_ref[...], preferred_element_type=jnp.float32)
        yn_ref[:N, :] = hw * dinv_ref[...]
        yn_ref[N:, :] = jnp.zeros((NPY - N, H), jnp.float32)

    return pl.pallas_call(
        body,
        out_shape=(
            jax.ShapeDtypeStruct((N, H), jnp.float32),
            jax.ShapeDtypeStruct((NPY, H), jnp.float32),
        ),
    )(acc, y, dinvM, g, be, wnt)


def _tc_res(acc, y, dinvM, g, be, res, wrest, bres, w4t):
    def body(acc_ref, y_ref, dinv_ref, g_ref, be_ref, res_ref, wr_ref, br_ref,
             w_ref, yn_ref):
        z = (acc_ref[0, :N, :] + acc_ref[1, :N, :] + y_ref[:N, :]) * dinv_ref[...]
        bn = _bn_relu(z, g_ref, be_ref)
        rw = jnp.dot(res_ref[...], wr_ref[...], preferred_element_type=jnp.float32)
        h = jnp.maximum(bn + rw + br_ref[...], 0.0)
        hw = jnp.dot(h, w_ref[...], preferred_element_type=jnp.float32)
        yn_ref[:N, :] = hw * dinv_ref[...]
        yn_ref[N:, :] = jnp.zeros((NPY - N, H), jnp.float32)

    return pl.pallas_call(
        body,
        out_shape=jax.ShapeDtypeStruct((NPY, H), jnp.float32),
    )(acc, y, dinvM, g, be, res, wrest, bres, w4t)


def _tc_head(acc, y, dinvM, g, be, wvt, bv, wot, bo, woutt, bout):
    def body(acc_ref, y_ref, dinv_ref, g_ref, be_ref, wv_ref, bv_ref, wo_ref,
             bo_ref, wout_ref, bout_ref, out_ref):
        z = (acc_ref[0, :N, :] + acc_ref[1, :N, :] + y_ref[:N, :]) * dinv_ref[...]
        h = jnp.maximum(_bn_relu(z, g_ref, be_ref), 0.0)
        v = jnp.dot(h, wv_ref[...], preferred_element_type=jnp.float32) + bv_ref[...]
        o = jnp.dot(v, wo_ref[...], preferred_element_type=jnp.float32) + bo_ref[...]
        out_ref[...] = (
            jnp.dot(o, wout_ref[...], preferred_element_type=jnp.float32)
            + bout_ref[...]
        )

    return pl.pallas_call(
        body,
        out_shape=jax.ShapeDtypeStruct((N, H), jnp.float32),
    )(acc, y, dinvM, g, be, wvt, bv, wot, bo, woutt, bout)


def kernel(x, edge_index, params):
    p = params
    pad = jnp.full((EPAD2 - E,), N, jnp.int32)
    srcf = jnp.concatenate([edge_index[0], pad]).reshape(NCHUNK, CH)
    dstf = jnp.concatenate([edge_index[1], pad]).reshape(NCHUNK, CH)
    sd = jnp.stack([srcf, dstf], axis=1)
    zeros_blk = jnp.zeros((SLC, H), jnp.float32)
    id_rows = jnp.arange(NR, dtype=jnp.int32).reshape(1, NR)

    def row(b):
        return b.reshape(1, H)

    degM = _sc_degree(sd, id_rows, zeros_blk).reshape(NC, NR * H, 1)
    dinvM, y1 = _tc_first(x, degM, p["W1"].T)
    acc1 = _sc_gather_scatter(y1, sd, zeros_blk)
    h1, y2 = _tc_mid(acc1, y1, dinvM, row(p["g1"]), row(p["be1"]), p["W2"].T)
    acc2 = _sc_gather_scatter(y2, sd, zeros_blk)
    _, y3 = _tc_mid(acc2, y2, dinvM, row(p["g2"]), row(p["be2"]), p["W3"].T)
    acc3 = _sc_gather_scatter(y3, sd, zeros_blk)
    y4 = _tc_res(acc3, y3, dinvM, row(p["g3"]), row(p["be3"]), h1,
                 p["Wres"].T, row(p["bres"]), p["W4"].T)
    acc4 = _sc_gather_scatter(y4, sd, zeros_blk)
    out = _tc_head(acc4, y4, dinvM, row(p["g4"]), row(p["be4"]),
                   p["Wv"].T, row(p["bv"]), p["Wo"].T, row(p["bo"]),
                   p["Wout"].T, row(p["bout"]))
    return out[None]


# speedup vs baseline: 1.8540x; 1.0101x over previous
import dataclasses
import functools

import jax
import jax.numpy as jnp
from jax import lax
from jax.experimental import pallas as pl
from jax.experimental.pallas import tpu as pltpu
from jax.experimental.pallas import tpu_sc as plsc

N = 10000
H = 128
E = 320000
EPS = 1e-5
NC = 2
NS = 16
NW = NC * NS
CH = 128
NSTEP0 = 120
NSTEP1 = 38
NCHUNK = NS * (NSTEP0 + NSTEP1)
C0TOT = NS * NSTEP0
EPAD2 = NCHUNK * CH
NSTEPD = NCHUNK // NW
NP = 10112
SLC = NP // NS
NPY = N + 16
NR = 80

_MESH = plsc.VectorSubcoreMesh(
    core_axis_name="c", subcore_axis_name="s", num_cores=NC, num_subcores=NS
)

_CP = pltpu.CompilerParams()
if "needs_layout_passes" in pltpu.CompilerParams.__dataclass_fields__:
    _CP = dataclasses.replace(_CP, needs_layout_passes=False)


def _sc_gather_scatter(y_pad, sd_idx, zeros_blk):
    @functools.partial(
        pl.kernel,
        out_type=jax.ShapeDtypeStruct((NC, NP, H), jnp.float32),
        mesh=_MESH,
        scratch_types=[
            pltpu.VMEM((2, CH), jnp.int32),
            pltpu.VMEM((2, CH), jnp.int32),
            pltpu.VMEM((CH, H), jnp.float32),
            pltpu.VMEM((CH, H), jnp.float32),
            pltpu.VMEM_SHARED((NP, H), jnp.float32),
            pltpu.SemaphoreType.DMA,
            pltpu.SemaphoreType.DMA,
            pltpu.SemaphoreType.DMA,
            pltpu.SemaphoreType.DMA,
            pltpu.SemaphoreType.DMA,
            pltpu.SemaphoreType.DMA,
        ],
    )
    def k(y_hbm, sd_hbm, z_hbm, acc_hbm, iA, iB, b0, b1, acc_sh,
          ia, ib, g0, g1, s0, s1):
        c = lax.axis_index("c")
        s = lax.axis_index("s")
        base = jnp.where(c == 0, s * NSTEP0, C0TOT + s * NSTEP1)
        pltpu.sync_copy(z_hbm, acc_sh.at[pl.ds(s * SLC, SLC)])

        def idx_load(i, buf, sem):
            pltpu.async_copy(sd_hbm.at[base + i], buf, sem)

        def wait_idx(buf, sem):
            pltpu.make_async_copy(sd_hbm.at[0], buf, sem).wait()

        def gather(buf_i, buf, sem):
            pltpu.async_copy(y_hbm.at[buf_i.at[0]], buf, sem)

        def wait_gather(buf, sem):
            pltpu.make_async_copy(y_hbm.at[pl.ds(0, CH)], buf, sem).wait()

        def scatter(buf_i, buf, sem):
            pltpu.async_copy(buf, acc_sh.at[buf_i.at[1]], sem, add=True)

        def wait_scatter(buf, sem):
            pltpu.make_async_copy(buf, acc_sh.at[pl.ds(0, CH)], sem).wait()

        nstep = jnp.where(c == 0, NSTEP0, NSTEP1)
        idx_load(0, iA, ia)
        idx_load(1, iB, ib)
        plsc.subcore_barrier()
        wait_idx(iA, ia)
        gather(iA, b0, g0)
        wait_idx(iB, ib)
        gather(iB, b1, g1)

        @pl.loop(0, NSTEP0, step=2)
        def _(i):
            @pl.when(i < nstep)
            def _():
                wait_gather(b0, g0)
                scatter(iA, b0, s0)
                wait_gather(b1, g1)
                scatter(iB, b1, s1)

                @pl.when(i + 2 < nstep)
                def _():
                    wait_scatter(b0, s0)
                    idx_load(i + 2, iA, ia)
                    wait_scatter(b1, s1)
                    idx_load(i + 3, iB, ib)
                    wait_idx(iA, ia)
                    gather(iA, b0, g0)
                    wait_idx(iB, ib)
                    gather(iB, b1, g1)

        wait_scatter(b0, s0)
        wait_scatter(b1, s1)
        plsc.subcore_barrier()
        pltpu.sync_copy(
            acc_sh.at[pl.ds(s * SLC, SLC)], acc_hbm.at[c, pl.ds(s * SLC, SLC)]
        )

    return k(y_pad, sd_idx, zeros_blk)


def _sc_degree(sd_idx, id_rows, zeros_blk):
    @functools.partial(
        pl.kernel,
        out_type=jax.ShapeDtypeStruct((NC, NR, H), jnp.float32),
        mesh=_MESH,
        compiler_params=_CP,
        scratch_types=[
            pltpu.VMEM((NSTEPD, 2, CH), jnp.int32),
            pltpu.VMEM((NR, H), jnp.float32),
            pltpu.VMEM((1, NR), jnp.int32),
            pltpu.VMEM_SHARED((NR, H), jnp.float32),
        ],
    )
    def k(sd_hbm, id_hbm, z_hbm, out_hbm, idx_all, hist, id_v, acc_sh):
        c = lax.axis_index("c")
        s = lax.axis_index("s")
        base = (c * NS + s) * NSTEPD
        pltpu.sync_copy(sd_hbm.at[pl.ds(base, NSTEPD)], idx_all)
        pltpu.sync_copy(id_hbm, id_v)

        @pl.when(s == 0)
        def _():
            pltpu.sync_copy(z_hbm.at[pl.ds(0, NR)], acc_sh)


        @pl.loop(0, NR)
        def _(r):
            for g in range(8):
                hist[r, pl.ds(16 * g, 16)] = jnp.zeros((16,), jnp.float32)

        ones16 = jnp.ones((16,), jnp.float32)

        @pl.loop(0, NSTEPD)
        def _(i):
            for g in range(8):
                d = idx_all[i, 1, pl.ds(16 * g, 16)]
                plsc.addupdate_scatter(hist, [d >> 7, d & 127], ones16)

        plsc.subcore_barrier()
        pltpu.sync_copy(hist, acc_sh.at[id_v.at[0]], add=True)
        plsc.subcore_barrier()

        @pl.when(s == 0)
        def _():
            pltpu.sync_copy(acc_sh, out_hbm.at[c])

    return k(sd_idx, id_rows, zeros_blk)


def _tc_first(x, degM, w1t):
    def body(x_ref, deg_ref, w_ref, dinv_ref, y_ref):
        deg = deg_ref[0, :N, :] + deg_ref[1, :N, :] + 1.0
        dinv = lax.rsqrt(deg) * jnp.ones((1, H), jnp.float32)
        dinv_ref[...] = dinv
        xw = jnp.dot(x_ref[...], w_ref[...], preferred_element_type=jnp.float32)
        y_ref[:N, :] = xw * dinv
        y_ref[N:, :] = jnp.zeros((NPY - N, H), jnp.float32)

    return pl.pallas_call(
        body,
        out_shape=(
            jax.ShapeDtypeStruct((N, H), jnp.float32),
            jax.ShapeDtypeStruct((NPY, H), jnp.float32),
        ),
    )(x, degM, w1t)


def _bn_relu(z, g_ref, be_ref):
    m = jnp.mean(z, axis=0, keepdims=True)
    zc = z - m
    v = jnp.mean(zc * zc, axis=0, keepdims=True)
    return zc * lax.rsqrt(v + EPS) * g_ref[...] + be_ref[...]


def _tc_mid(acc, y, dinvM, g, be, wnt):
    def body(acc_ref, y_ref, dinv_ref, g_ref, be_ref, w_ref, h_ref, yn_ref):
        z = (acc_ref[0, :N, :] + acc_ref[1, :N, :] + y_ref[:N, :]) * dinv_ref[...]
        h = jnp.maximum(_bn_relu(z, g_ref, be_ref), 0.0)
        h_ref[...] = h
        hw = jnp.dot(h, w_ref[...], preferred_element_type=jnp.float32)
        yn_ref[:N, :] = hw * dinv_ref[...]
        yn_ref[N:, :] = jnp.zeros((NPY - N, H), jnp.float32)

    return pl.pallas_call(
        body,
        out_shape=(
            jax.ShapeDtypeStruct((N, H), jnp.float32),
            jax.ShapeDtypeStruct((NPY, H), jnp.float32),
        ),
    )(acc, y, dinvM, g, be, wnt)


def _tc_res(acc, y, dinvM, g, be, res, wrest, bres, w4t):
    def body(acc_ref, y_ref, dinv_ref, g_ref, be_ref, res_ref, wr_ref, br_ref,
             w_ref, yn_ref):
        z = (acc_ref[0, :N, :] + acc_ref[1, :N, :] + y_ref[:N, :]) * dinv_ref[...]
        bn = _bn_relu(z, g_ref, be_ref)
        rw = jnp.dot(res_ref[...], wr_ref[...], preferred_element_type=jnp.float32)
        h = jnp.maximum(bn + rw + br_ref[...], 0.0)
        hw = jnp.dot(h, w_ref[...], preferred_element_type=jnp.float32)
        yn_ref[:N, :] = hw * dinv_ref[...]
        yn_ref[N:, :] = jnp.zeros((NPY - N, H), jnp.float32)

    return pl.pallas_call(
        body,
        out_shape=jax.ShapeDtypeStruct((NPY, H), jnp.float32),
    )(acc, y, dinvM, g, be, res, wrest, bres, w4t)


def _tc_head(acc, y, dinvM, g, be, wvt, bv, wot, bo, woutt, bout):
    def body(acc_ref, y_ref, dinv_ref, g_ref, be_ref, wv_ref, bv_ref, wo_ref,
             bo_ref, wout_ref, bout_ref, out_ref):
        z = (acc_ref[0, :N, :] + acc_ref[1, :N, :] + y_ref[:N, :]) * dinv_ref[...]
        h = jnp.maximum(_bn_relu(z, g_ref, be_ref), 0.0)
        v = jnp.dot(h, wv_ref[...], preferred_element_type=jnp.float32) + bv_ref[...]
        o = jnp.dot(v, wo_ref[...], preferred_element_type=jnp.float32) + bo_ref[...]
        out_ref[...] = (
            jnp.dot(o, wout_ref[...], preferred_element_type=jnp.float32)
            + bout_ref[...]
        )

    return pl.pallas_call(
        body,
        out_shape=jax.ShapeDtypeStruct((N, H), jnp.float32),
    )(acc, y, dinvM, g, be, wvt, bv, wot, bo, woutt, bout)


def kernel(x, edge_index, params):
    p = params
    pad = jnp.full((EPAD2 - E,), N, jnp.int32)
    srcf = jnp.concatenate([edge_index[0], pad]).reshape(NCHUNK, CH)
    dstf = jnp.concatenate([edge_index[1], pad]).reshape(NCHUNK, CH)
    sd = jnp.stack([srcf, dstf], axis=1)
    zeros_blk = jnp.zeros((SLC, H), jnp.float32)
    id_rows = jnp.arange(NR, dtype=jnp.int32).reshape(1, NR)

    def row(b):
        return b.reshape(1, H)

    degM = _sc_degree(sd, id_rows, zeros_blk).reshape(NC, NR * H, 1)
    dinvM, y1 = _tc_first(x, degM, p["W1"].T)
    acc1 = _sc_gather_scatter(y1, sd, zeros_blk)
    h1, y2 = _tc_mid(acc1, y1, dinvM, row(p["g1"]), row(p["be1"]), p["W2"].T)
    acc2 = _sc_gather_scatter(y2, sd, zeros_blk)
    _, y3 = _tc_mid(acc2, y2, dinvM, row(p["g2"]), row(p["be2"]), p["W3"].T)
    acc3 = _sc_gather_scatter(y3, sd, zeros_blk)
    y4 = _tc_res(acc3, y3, dinvM, row(p["g3"]), row(p["be3"]), h1,
                 p["Wres"].T, row(p["bres"]), p["W4"].T)
    acc4 = _sc_gather_scatter(y4, sd, zeros_blk)
    out = _tc_head(acc4, y4, dinvM, row(p["g4"]), row(p["be4"]),
                   p["Wv"].T, row(p["bv"]), p["Wo"].T, row(p["bo"]),
                   p["Wout"].T, row(p["bout"]))
    return out[None]


# speedup vs baseline: 1.8552x; 1.0007x over previous
import dataclasses
import functools

import jax
import jax.numpy as jnp
from jax import lax
from jax.experimental import pallas as pl
from jax.experimental.pallas import tpu as pltpu
from jax.experimental.pallas import tpu_sc as plsc

N = 10000
H = 128
E = 320000
EPS = 1e-5
NC = 2
NS = 16
NW = NC * NS
CH = 128
NSTEP0 = 120
NSTEP1 = 38
NCHUNK = NS * (NSTEP0 + NSTEP1)
C0TOT = NS * NSTEP0
EPAD2 = NCHUNK * CH
NSTEPD = NCHUNK // NW
NP = 10112
SLC = NP // NS
NPY = N + 16
NR = 80

_MESH = plsc.VectorSubcoreMesh(
    core_axis_name="c", subcore_axis_name="s", num_cores=NC, num_subcores=NS
)

_CP = pltpu.CompilerParams()
if "needs_layout_passes" in pltpu.CompilerParams.__dataclass_fields__:
    _CP = dataclasses.replace(_CP, needs_layout_passes=False)


def _sc_gather_scatter(y_pad, sd_idx, zeros_blk):
    @functools.partial(
        pl.kernel,
        out_type=jax.ShapeDtypeStruct((NC, NP, H), jnp.float32),
        mesh=_MESH,
        scratch_types=[
            pltpu.VMEM((2, CH), jnp.int32),
            pltpu.VMEM((2, CH), jnp.int32),
            pltpu.VMEM((CH, H), jnp.float32),
            pltpu.VMEM((CH, H), jnp.float32),
            pltpu.VMEM_SHARED((NP, H), jnp.float32),
            pltpu.SemaphoreType.DMA,
            pltpu.SemaphoreType.DMA,
            pltpu.SemaphoreType.DMA,
            pltpu.SemaphoreType.DMA,
            pltpu.SemaphoreType.DMA,
            pltpu.SemaphoreType.DMA,
        ],
    )
    def k(y_hbm, sd_hbm, z_hbm, acc_hbm, iA, iB, b0, b1, acc_sh,
          ia, ib, g0, g1, s0, s1):
        c = lax.axis_index("c")
        s = lax.axis_index("s")
        base = jnp.where(c == 0, s * NSTEP0, C0TOT + s * NSTEP1)
        pltpu.sync_copy(z_hbm, acc_sh.at[pl.ds(s * SLC, SLC)])

        def idx_load(i, buf, sem):
            pltpu.async_copy(sd_hbm.at[base + i], buf, sem)

        def wait_idx(buf, sem):
            pltpu.make_async_copy(sd_hbm.at[0], buf, sem).wait()

        def gather(buf_i, buf, sem):
            pltpu.async_copy(
                y_hbm.at[buf_i.at[0, pl.ds(0, 64)]], buf.at[pl.ds(0, 64)], sem
            )
            pltpu.async_copy(
                y_hbm.at[buf_i.at[0, pl.ds(64, 64)]], buf.at[pl.ds(64, 64)], sem
            )

        def wait_gather(buf, sem):
            pltpu.make_async_copy(y_hbm.at[pl.ds(0, CH)], buf, sem).wait()

        def scatter(buf_i, buf, sem):
            pltpu.async_copy(buf, acc_sh.at[buf_i.at[1]], sem, add=True)

        def wait_scatter(buf, sem):
            pltpu.make_async_copy(buf, acc_sh.at[pl.ds(0, CH)], sem).wait()

        nstep = jnp.where(c == 0, NSTEP0, NSTEP1)
        idx_load(0, iA, ia)
        idx_load(1, iB, ib)
        plsc.subcore_barrier()
        wait_idx(iA, ia)
        gather(iA, b0, g0)
        wait_idx(iB, ib)
        gather(iB, b1, g1)

        @pl.loop(0, NSTEP0, step=2)
        def _(i):
            @pl.when(i < nstep)
            def _():
                wait_gather(b0, g0)
                scatter(iA, b0, s0)
                wait_gather(b1, g1)
                scatter(iB, b1, s1)

                @pl.when(i + 2 < nstep)
                def _():
                    wait_scatter(b0, s0)
                    idx_load(i + 2, iA, ia)
                    wait_scatter(b1, s1)
                    idx_load(i + 3, iB, ib)
                    wait_idx(iA, ia)
                    gather(iA, b0, g0)
                    wait_idx(iB, ib)
                    gather(iB, b1, g1)

        wait_scatter(b0, s0)
        wait_scatter(b1, s1)
        plsc.subcore_barrier()
        pltpu.sync_copy(
            acc_sh.at[pl.ds(s * SLC, SLC)], acc_hbm.at[c, pl.ds(s * SLC, SLC)]
        )

    return k(y_pad, sd_idx, zeros_blk)


def _sc_degree(sd_idx, id_rows, zeros_blk):
    @functools.partial(
        pl.kernel,
        out_type=jax.ShapeDtypeStruct((NC, NR, H), jnp.float32),
        mesh=_MESH,
        compiler_params=_CP,
        scratch_types=[
            pltpu.VMEM((NSTEPD, 2, CH), jnp.int32),
            pltpu.VMEM((NR, H), jnp.float32),
            pltpu.VMEM((1, NR), jnp.int32),
            pltpu.VMEM_SHARED((NR, H), jnp.float32),
        ],
    )
    def k(sd_hbm, id_hbm, z_hbm, out_hbm, idx_all, hist, id_v, acc_sh):
        c = lax.axis_index("c")
        s = lax.axis_index("s")
        base = (c * NS + s) * NSTEPD
        pltpu.sync_copy(sd_hbm.at[pl.ds(base, NSTEPD)], idx_all)
        pltpu.sync_copy(id_hbm, id_v)

        @pl.when(s == 0)
        def _():
            pltpu.sync_copy(z_hbm.at[pl.ds(0, NR)], acc_sh)


        @pl.loop(0, NR)
        def _(r):
            for g in range(8):
                hist[r, pl.ds(16 * g, 16)] = jnp.zeros((16,), jnp.float32)

        ones16 = jnp.ones((16,), jnp.float32)

        @pl.loop(0, NSTEPD)
        def _(i):
            for g in range(8):
                d = idx_all[i, 1, pl.ds(16 * g, 16)]
                plsc.addupdate_scatter(hist, [d >> 7, d & 127], ones16)

        plsc.subcore_barrier()
        pltpu.sync_copy(hist, acc_sh.at[id_v.at[0]], add=True)
        plsc.subcore_barrier()

        @pl.when(s == 0)
        def _():
            pltpu.sync_copy(acc_sh, out_hbm.at[c])

    return k(sd_idx, id_rows, zeros_blk)


def _tc_first(x, degM, w1t):
    def body(x_ref, deg_ref, w_ref, dinv_ref, y_ref):
        deg = deg_ref[0, :N, :] + deg_ref[1, :N, :] + 1.0
        dinv = lax.rsqrt(deg) * jnp.ones((1, H), jnp.float32)
        dinv_ref[...] = dinv
        xw = jnp.dot(x_ref[...], w_ref[...], preferred_element_type=jnp.float32)
        y_ref[:N, :] = xw * dinv
        y_ref[N:, :] = jnp.zeros((NPY - N, H), jnp.float32)

    return pl.pallas_call(
        body,
        out_shape=(
            jax.ShapeDtypeStruct((N, H), jnp.float32),
            jax.ShapeDtypeStruct((NPY, H), jnp.float32),
        ),
    )(x, degM, w1t)


def _bn_relu(z, g_ref, be_ref):
    m = jnp.mean(z, axis=0, keepdims=True)
    zc = z - m
    v = jnp.mean(zc * zc, axis=0, keepdims=True)
    return zc * lax.rsqrt(v + EPS) * g_ref[...] + be_ref[...]


def _tc_mid(acc, y, dinvM, g, be, wnt):
    def body(acc_ref, y_ref, dinv_ref, g_ref, be_ref, w_ref, h_ref, yn_ref):
        z = (acc_ref[0, :N, :] + acc_ref[1, :N, :] + y_ref[:N, :]) * dinv_ref[...]
        h = jnp.maximum(_bn_relu(z, g_ref, be_ref), 0.0)
        h_ref[...] = h
        hw = jnp.dot(h, w_ref[...], preferred_element_type=jnp.float32)
        yn_ref[:N, :] = hw * dinv_ref[...]
        yn_ref[N:, :] = jnp.zeros((NPY - N, H), jnp.float32)

    return pl.pallas_call(
        body,
        out_shape=(
            jax.ShapeDtypeStruct((N, H), jnp.float32),
            jax.ShapeDtypeStruct((NPY, H), jnp.float32),
        ),
    )(acc, y, dinvM, g, be, wnt)


def _tc_res(acc, y, dinvM, g, be, res, wrest, bres, w4t):
    def body(acc_ref, y_ref, dinv_ref, g_ref, be_ref, res_ref, wr_ref, br_ref,
             w_ref, yn_ref):
        z = (acc_ref[0, :N, :] + acc_ref[1, :N, :] + y_ref[:N, :]) * dinv_ref[...]
        bn = _bn_relu(z, g_ref, be_ref)
        rw = jnp.dot(res_ref[...], wr_ref[...], preferred_element_type=jnp.float32)
        h = jnp.maximum(bn + rw + br_ref[...], 0.0)
        hw = jnp.dot(h, w_ref[...], preferred_element_type=jnp.float32)
        yn_ref[:N, :] = hw * dinv_ref[...]
        yn_ref[N:, :] = jnp.zeros((NPY - N, H), jnp.float32)

    return pl.pallas_call(
        body,
        out_shape=jax.ShapeDtypeStruct((NPY, H), jnp.float32),
    )(acc, y, dinvM, g, be, res, wrest, bres, w4t)


def _tc_head(acc, y, dinvM, g, be, wvt, bv, wot, bo, woutt, bout):
    def body(acc_ref, y_ref, dinv_ref, g_ref, be_ref, wv_ref, bv_ref, wo_ref,
             bo_ref, wout_ref, bout_ref, out_ref):
        z = (acc_ref[0, :N, :] + acc_ref[1, :N, :] + y_ref[:N, :]) * dinv_ref[...]
        h = jnp.maximum(_bn_relu(z, g_ref, be_ref), 0.0)
        v = jnp.dot(h, wv_ref[...], preferred_element_type=jnp.float32) + bv_ref[...]
        o = jnp.dot(v, wo_ref[...], preferred_element_type=jnp.float32) + bo_ref[...]
        out_ref[...] = (
            jnp.dot(o, wout_ref[...], preferred_element_type=jnp.float32)
            + bout_ref[...]
        )

    return pl.pallas_call(
        body,
        out_shape=jax.ShapeDtypeStruct((N, H), jnp.float32),
    )(acc, y, dinvM, g, be, wvt, bv, wot, bo, woutt, bout)


def kernel(x, edge_index, params):
    p = params
    pad = jnp.full((EPAD2 - E,), N, jnp.int32)
    srcf = jnp.concatenate([edge_index[0], pad]).reshape(NCHUNK, CH)
    dstf = jnp.concatenate([edge_index[1], pad]).reshape(NCHUNK, CH)
    sd = jnp.stack([srcf, dstf], axis=1)
    zeros_blk = jnp.zeros((SLC, H), jnp.float32)
    id_rows = jnp.arange(NR, dtype=jnp.int32).reshape(1, NR)

    def row(b):
        return b.reshape(1, H)

    degM = _sc_degree(sd, id_rows, zeros_blk).reshape(NC, NR * H, 1)
    dinvM, y1 = _tc_first(x, degM, p["W1"].T)
    acc1 = _sc_gather_scatter(y1, sd, zeros_blk)
    h1, y2 = _tc_mid(acc1, y1, dinvM, row(p["g1"]), row(p["be1"]), p["W2"].T)
    acc2 = _sc_gather_scatter(y2, sd, zeros_blk)
    _, y3 = _tc_mid(acc2, y2, dinvM, row(p["g2"]), row(p["be2"]), p["W3"].T)
    acc3 = _sc_gather_scatter(y3, sd, zeros_blk)
    y4 = _tc_res(acc3, y3, dinvM, row(p["g3"]), row(p["be3"]), h1,
                 p["Wres"].T, row(p["bres"]), p["W4"].T)
    acc4 = _sc_gather_scatter(y4, sd, zeros_blk)
    out = _tc_head(acc4, y4, dinvM, row(p["g4"]), row(p["be4"]),
                   p["Wv"].T, row(p["bv"]), p["Wo"].T, row(p["bo"]),
                   p["Wout"].T, row(p["bout"]))
    return out[None]


# speedup vs baseline: 1.8553x; 1.0000x over previous
import dataclasses
import functools

import jax
import jax.numpy as jnp
from jax import lax
from jax.experimental import pallas as pl
from jax.experimental.pallas import tpu as pltpu
from jax.experimental.pallas import tpu_sc as plsc

N = 10000
H = 128
E = 320000
EPS = 1e-5
NC = 2
NS = 16
NW = NC * NS
CH = 128
NSTEP0 = 120
NSTEP1 = 38
NCHUNK = NS * (NSTEP0 + NSTEP1)
C0TOT = NS * NSTEP0
EPAD2 = NCHUNK * CH
NSTEPD = NCHUNK // NW
NP = 10112
SLC = NP // NS
NPY = N + 16
NR = 80

_MESH = plsc.VectorSubcoreMesh(
    core_axis_name="c", subcore_axis_name="s", num_cores=NC, num_subcores=NS
)

_CP = pltpu.CompilerParams()
if "needs_layout_passes" in pltpu.CompilerParams.__dataclass_fields__:
    _CP = dataclasses.replace(_CP, needs_layout_passes=False)


def _sc_gather_scatter(y_pad, sd_idx, zeros_blk):
    @functools.partial(
        pl.kernel,
        out_type=jax.ShapeDtypeStruct((NC, NP, H), jnp.float32),
        mesh=_MESH,
        scratch_types=[
            pltpu.VMEM((2, CH), jnp.int32),
            pltpu.VMEM((2, CH), jnp.int32),
            pltpu.VMEM((CH, H), jnp.float32),
            pltpu.VMEM((CH, H), jnp.float32),
            pltpu.VMEM_SHARED((NP, H), jnp.float32),
            pltpu.SemaphoreType.DMA,
            pltpu.SemaphoreType.DMA,
            pltpu.SemaphoreType.DMA,
            pltpu.SemaphoreType.DMA,
            pltpu.SemaphoreType.DMA,
            pltpu.SemaphoreType.DMA,
        ],
    )
    def k(y_hbm, sd_hbm, z_hbm, acc_hbm, iA, iB, b0, b1, acc_sh,
          ia, ib, g0, g1, s0, s1):
        c = lax.axis_index("c")
        s = lax.axis_index("s")
        base = jnp.where(c == 0, s * NSTEP0, C0TOT + s * NSTEP1)
        pltpu.sync_copy(z_hbm, acc_sh.at[pl.ds(s * SLC, SLC)])

        def idx_load(i, buf, sem):
            pltpu.async_copy(sd_hbm.at[base + i], buf, sem)

        def wait_idx(buf, sem):
            pltpu.make_async_copy(sd_hbm.at[0], buf, sem).wait()

        def gather(buf_i, buf, sem):
            pltpu.async_copy(y_hbm.at[buf_i.at[0]], buf, sem)

        def wait_gather(buf, sem):
            pltpu.make_async_copy(y_hbm.at[pl.ds(0, CH)], buf, sem).wait()

        def scatter(buf_i, buf, sem):
            pltpu.async_copy(buf, acc_sh.at[buf_i.at[1]], sem, add=True)

        def wait_scatter(buf, sem):
            pltpu.make_async_copy(buf, acc_sh.at[pl.ds(0, CH)], sem).wait()

        nstep = jnp.where(c == 0, NSTEP0, NSTEP1)
        idx_load(0, iA, ia)
        idx_load(1, iB, ib)
        plsc.subcore_barrier()
        wait_idx(iA, ia)
        gather(iA, b0, g0)
        wait_idx(iB, ib)
        gather(iB, b1, g1)

        @pl.loop(0, NSTEP0, step=2)
        def _(i):
            @pl.when(i < nstep)
            def _():
                wait_gather(b0, g0)
                scatter(iA, b0, s0)
                wait_gather(b1, g1)
                scatter(iB, b1, s1)

                @pl.when(i + 2 < nstep)
                def _():
                    wait_scatter(b0, s0)
                    idx_load(i + 2, iA, ia)
                    wait_scatter(b1, s1)
                    idx_load(i + 3, iB, ib)
                    wait_idx(iA, ia)
                    gather(iA, b0, g0)
                    wait_idx(iB, ib)
                    gather(iB, b1, g1)

        wait_scatter(b0, s0)
        wait_scatter(b1, s1)
        plsc.subcore_barrier()
        pltpu.sync_copy(
            acc_sh.at[pl.ds(s * SLC, SLC)], acc_hbm.at[c, pl.ds(s * SLC, SLC)]
        )

    return k(y_pad, sd_idx, zeros_blk)


def _sc_degree(sd_idx, id_rows, zeros_blk):
    @functools.partial(
        pl.kernel,
        out_type=jax.ShapeDtypeStruct((NC, NR, H), jnp.float32),
        mesh=_MESH,
        compiler_params=_CP,
        scratch_types=[
            pltpu.VMEM((NSTEPD, 2, CH), jnp.int32),
            pltpu.VMEM((NR, H), jnp.float32),
            pltpu.VMEM((1, NR), jnp.int32),
            pltpu.VMEM_SHARED((NR, H), jnp.float32),
        ],
    )
    def k(sd_hbm, id_hbm, z_hbm, out_hbm, idx_all, hist, id_v, acc_sh):
        c = lax.axis_index("c")
        s = lax.axis_index("s")
        base = (c * NS + s) * NSTEPD
        pltpu.sync_copy(sd_hbm.at[pl.ds(base, NSTEPD)], idx_all)
        pltpu.sync_copy(id_hbm, id_v)

        @pl.when(s == 0)
        def _():
            pltpu.sync_copy(z_hbm.at[pl.ds(0, NR)], acc_sh)


        @pl.loop(0, NR)
        def _(r):
            for g in range(8):
                hist[r, pl.ds(16 * g, 16)] = jnp.zeros((16,), jnp.float32)

        ones16 = jnp.ones((16,), jnp.float32)

        @pl.loop(0, NSTEPD)
        def _(i):
            for g in range(8):
                d = idx_all[i, 1, pl.ds(16 * g, 16)]
                plsc.addupdate_scatter(hist, [d >> 7, d & 127], ones16)

        plsc.subcore_barrier()
        pltpu.sync_copy(hist, acc_sh.at[id_v.at[0]], add=True)
        plsc.subcore_barrier()

        @pl.when(s == 0)
        def _():
            pltpu.sync_copy(acc_sh, out_hbm.at[c])

    return k(sd_idx, id_rows, zeros_blk)


def _tc_first(x, degM, w1t):
    def body(x_ref, deg_ref, w_ref, dinv_ref, y_ref):
        deg = deg_ref[0, :N, :] + deg_ref[1, :N, :] + 1.0
        dinv = lax.rsqrt(deg) * jnp.ones((1, H), jnp.float32)
        dinv_ref[...] = dinv
        xw = jnp.dot(x_ref[...], w_ref[...], preferred_element_type=jnp.float32)
        y_ref[:N, :] = xw * dinv
        y_ref[N:, :] = jnp.zeros((NPY - N, H), jnp.float32)

    return pl.pallas_call(
        body,
        out_shape=(
            jax.ShapeDtypeStruct((N, H), jnp.float32),
            jax.ShapeDtypeStruct((NPY, H), jnp.float32),
        ),
    )(x, degM, w1t)


def _bn_relu(z, g_ref, be_ref):
    m = jnp.mean(z, axis=0, keepdims=True)
    zc = z - m
    v = jnp.mean(zc * zc, axis=0, keepdims=True)
    return zc * lax.rsqrt(v + EPS) * g_ref[...] + be_ref[...]


def _tc_mid(acc, y, dinvM, g, be, wnt):
    def body(acc_ref, y_ref, dinv_ref, g_ref, be_ref, w_ref, h_ref, yn_ref):
        z = (acc_ref[0, :N, :] + acc_ref[1, :N, :] + y_ref[:N, :]) * dinv_ref[...]
        h = jnp.maximum(_bn_relu(z, g_ref, be_ref), 0.0)
        h_ref[...] = h
        hw = jnp.dot(h, w_ref[...], preferred_element_type=jnp.float32)
        yn_ref[:N, :] = hw * dinv_ref[...]
        yn_ref[N:, :] = jnp.zeros((NPY - N, H), jnp.float32)

    return pl.pallas_call(
        body,
        out_shape=(
            jax.ShapeDtypeStruct((N, H), jnp.float32),
            jax.ShapeDtypeStruct((NPY, H), jnp.float32),
        ),
    )(acc, y, dinvM, g, be, wnt)


def _tc_res(acc, y, dinvM, g, be, res, wrest, bres, w4t):
    def body(acc_ref, y_ref, dinv_ref, g_ref, be_ref, res_ref, wr_ref, br_ref,
             w_ref, yn_ref):
        z = (acc_ref[0, :N, :] + acc_ref[1, :N, :] + y_ref[:N, :]) * dinv_ref[...]
        bn = _bn_relu(z, g_ref, be_ref)
        rw = jnp.dot(res_ref[...], wr_ref[...], preferred_element_type=jnp.float32)
        h = jnp.maximum(bn + rw + br_ref[...], 0.0)
        hw = jnp.dot(h, w_ref[...], preferred_element_type=jnp.float32)
        yn_ref[:N, :] = hw * dinv_ref[...]
        yn_ref[N:, :] = jnp.zeros((NPY - N, H), jnp.float32)

    return pl.pallas_call(
        body,
        out_shape=jax.ShapeDtypeStruct((NPY, H), jnp.float32),
    )(acc, y, dinvM, g, be, res, wrest, bres, w4t)


def _tc_head(acc, y, dinvM, g, be, wvt, bv, wot, bo, woutt, bout):
    def body(acc_ref, y_ref, dinv_ref, g_ref, be_ref, wv_ref, bv_ref, wo_ref,
             bo_ref, wout_ref, bout_ref, out_ref):
        z = (acc_ref[0, :N, :] + acc_ref[1, :N, :] + y_ref[:N, :]) * dinv_ref[...]
        h = jnp.maximum(_bn_relu(z, g_ref, be_ref), 0.0)
        v = jnp.dot(h, wv_ref[...], preferred_element_type=jnp.float32) + bv_ref[...]
        o = jnp.dot(v, wo_ref[...], preferred_element_type=jnp.float32) + bo_ref[...]
        out_ref[...] = (
            jnp.dot(o, wout_ref[...], preferred_element_type=jnp.float32)
            + bout_ref[...]
        )

    return pl.pallas_call(
        body,
        out_shape=jax.ShapeDtypeStruct((N, H), jnp.float32),
    )(acc, y, dinvM, g, be, wvt, bv, wot, bo, woutt, bout)


def kernel(x, edge_index, params):
    p = params
    pad = jnp.full((EPAD2 - E,), N, jnp.int32)
    srcf = jnp.concatenate([edge_index[0], pad]).reshape(NCHUNK, CH)
    dstf = jnp.concatenate([edge_index[1], pad]).reshape(NCHUNK, CH)
    sd = jnp.stack([srcf, dstf], axis=1)
    zeros_blk = jnp.zeros((SLC, H), jnp.float32)
    id_rows = jnp.arange(NR, dtype=jnp.int32).reshape(1, NR)

    def row(b):
        return b.reshape(1, H)

    degM = _sc_degree(sd, id_rows, zeros_blk).reshape(NC, NR * H, 1)
    dinvM, y1 = _tc_first(x, degM, p["W1"].T)
    acc1 = _sc_gather_scatter(y1, sd, zeros_blk)
    h1, y2 = _tc_mid(acc1, y1, dinvM, row(p["g1"]), row(p["be1"]), p["W2"].T)
    acc2 = _sc_gather_scatter(y2, sd, zeros_blk)
    _, y3 = _tc_mid(acc2, y2, dinvM, row(p["g2"]), row(p["be2"]), p["W3"].T)
    acc3 = _sc_gather_scatter(y3, sd, zeros_blk)
    y4 = _tc_res(acc3, y3, dinvM, row(p["g3"]), row(p["be3"]), h1,
                 p["Wres"].T, row(p["bres"]), p["W4"].T)
    acc4 = _sc_gather_scatter(y4, sd, zeros_blk)
    out = _tc_head(acc4, y4, dinvM, row(p["g4"]), row(p["be4"]),
                   p["Wv"].T, row(p["bv"]), p["Wo"].T, row(p["bo"]),
                   p["Wout"].T, row(p["bout"]))
    return out[None]
